# Initial kernel scaffold; baseline (speedup 1.0000x reference)
import jax
import jax.numpy as jnp
from jax.experimental import pallas as pl


def kernel(theta, t, Y_train):
    raise NotImplementedError("write your pallas kernel here")


# speedup vs baseline: 5.4273x; 5.4273x over previous
import functools

import jax
import jax.numpy as jnp
from jax import lax
from jax.experimental import pallas as pl
from jax.experimental.pallas import tpu as pltpu
from jax.experimental.pallas import tpu_sc as plsc

B = 128
N = 32768
L = 16
NC = 2
NS = 16
NW = NC * NS
ROWS_PER_W = B // NW
CHUNK = 2048
SUB = 128


def _count_below(rowbuf, t_r):
    lane = lax.iota(jnp.int32, L)
    zero = jnp.zeros((L,), jnp.float32)

    def chunk_body(c, sums):
        base = c * CHUNK

        def vbody(i, accs):
            a0, a1, a2, a3 = accs
            off = base + i * (4 * L)
            a0 = a0 + rowbuf[pl.ds(off + 0 * L, L)]
            a1 = a1 + rowbuf[pl.ds(off + 1 * L, L)]
            a2 = a2 + rowbuf[pl.ds(off + 2 * L, L)]
            a3 = a3 + rowbuf[pl.ds(off + 3 * L, L)]
            return (a0, a1, a2, a3)

        a0, a1, a2, a3 = lax.fori_loop(
            0, CHUNK // (4 * L), vbody, (zero, zero, zero, zero))
        total = jnp.sum((a0 + a1) + (a2 + a3))
        return jnp.where(lane == c, total, sums)

    sums0 = lax.fori_loop(0, N // CHUNK, chunk_body, zero)
    scan0 = plsc.cumsum(sums0)
    mask0 = scan0 < t_r
    c_star = jnp.sum(mask0.astype(jnp.int32))
    s_before = jnp.sum(jnp.where(mask0, sums0, 0.0))
    count = c_star * CHUNK
    base1 = jnp.minimum(c_star, N // CHUNK - 1) * CHUNK

    def sub_body(s, sums):
        base = base1 + s * SUB

        def vbody(i, accs):
            a0, a1 = accs
            off = base + i * (2 * L)
            a0 = a0 + rowbuf[pl.ds(off + 0 * L, L)]
            a1 = a1 + rowbuf[pl.ds(off + 1 * L, L)]
            return (a0, a1)

        a0, a1 = lax.fori_loop(0, SUB // (2 * L), vbody, (zero, zero))
        total = jnp.sum(a0 + a1)
        return jnp.where(lane == s, total, sums)

    sums1 = lax.fori_loop(0, CHUNK // SUB, sub_body, zero)
    scan1 = plsc.cumsum(sums1)
    mask1 = (s_before + scan1) < t_r
    s_star = jnp.sum(mask1.astype(jnp.int32))
    s_before = s_before + jnp.sum(jnp.where(mask1, sums1, 0.0))
    count = count + s_star * SUB
    base2 = base1 + jnp.minimum(s_star, CHUNK // SUB - 1) * SUB

    def fine_body(i, carry):
        cnt, s_run = carry
        v = rowbuf[pl.ds(base2 + i * L, L)]
        sc = plsc.cumsum(v)
        cnt = cnt + jnp.sum(((s_run + sc) < t_r).astype(jnp.int32))
        s_run = s_run + jnp.sum(v)
        return (cnt, s_run)

    count, _ = lax.fori_loop(0, SUB // L, fine_body, (count, s_before))
    return count


def _sc_body(theta_hbm, t_hbm, y_hbm, out_hbm,
             t_v, y_v, rb0, rb1, val_v, sem0, sem1, sem_y):
    wid = lax.axis_index("s") * NC + lax.axis_index("c")
    row_base = wid * ROWS_PER_W
    lane = lax.iota(jnp.int32, L)

    y_cp = pltpu.make_async_copy(y_hbm, y_v, sem_y)
    y_cp.start()
    pltpu.sync_copy(t_hbm, t_v.at[pl.ds(0, B)])

    bufs = (rb0, rb1)
    sems = (sem0, sem1)
    cps = [None, None]
    cps[0] = pltpu.make_async_copy(theta_hbm.at[row_base], rb0, sem0)
    cps[0].start()

    idx_vec = jnp.zeros((L,), jnp.int32)
    for r in range(ROWS_PER_W):
        cur = r % 2
        cps[cur].wait()
        if r + 1 < ROWS_PER_W:
            nxt = (r + 1) % 2
            cps[nxt] = pltpu.make_async_copy(
                theta_hbm.at[row_base + r + 1], bufs[nxt], sems[nxt])
            cps[nxt].start()
        t_r = t_v[pl.ds(row_base + r, L)][0]
        count = _count_below(bufs[cur], t_r)
        idx = jnp.clip(count, 0, N - 1)
        idx_vec = jnp.where(lane == r, idx, idx_vec)

    y_cp.wait()
    val_v[...] = plsc.load_gather(y_v, [idx_vec])
    pltpu.sync_copy(val_v, out_hbm.at[wid])


@jax.jit
def _sc_quantile(theta, t, Y_train):
    mesh = plsc.VectorSubcoreMesh(core_axis_name="c", subcore_axis_name="s")
    out2d = pl.kernel(
        _sc_body,
        out_type=jax.ShapeDtypeStruct((NW, L), jnp.float32),
        mesh=mesh,
        compiler_params=pltpu.CompilerParams(needs_layout_passes=False),
        scratch_types=[
            pltpu.VMEM((B + L,), jnp.float32),
            pltpu.VMEM((N,), jnp.float32),
            pltpu.VMEM((N,), jnp.float32),
            pltpu.VMEM((N,), jnp.float32),
            pltpu.VMEM((L,), jnp.float32),
            pltpu.SemaphoreType.DMA,
            pltpu.SemaphoreType.DMA,
            pltpu.SemaphoreType.DMA,
        ],
    )(theta, t, Y_train)
    return out2d[:, :ROWS_PER_W].reshape(B)


def kernel(theta, t, Y_train):
    return _sc_quantile(theta, t, Y_train)


# speedup vs baseline: 6.2931x; 1.1595x over previous
import functools

import jax
import jax.numpy as jnp
from jax import lax
from jax.experimental import pallas as pl
from jax.experimental.pallas import tpu as pltpu
from jax.experimental.pallas import tpu_sc as plsc

B = 128
N = 32768
L = 16
NC = 2
NS = 16
NW = NC * NS
ROWS_PER_W = B // NW
CHUNK = 2048
SUB = 128


def _count_below(rowbuf, t_r):
    lane = lax.iota(jnp.int32, L)
    zero = jnp.zeros((L,), jnp.float32)

    def chunk_body(c, sums):
        base = c * CHUNK

        @plsc.parallel_loop(0, CHUNK // (4 * L), unroll=8,
                            carry=(zero, zero, zero, zero))
        def accs(i, carry):
            a0, a1, a2, a3 = carry
            off = base + i * (4 * L)
            a0 = a0 + rowbuf[pl.ds(off + 0 * L, L)]
            a1 = a1 + rowbuf[pl.ds(off + 1 * L, L)]
            a2 = a2 + rowbuf[pl.ds(off + 2 * L, L)]
            a3 = a3 + rowbuf[pl.ds(off + 3 * L, L)]
            return (a0, a1, a2, a3)

        a0, a1, a2, a3 = accs
        total = jnp.sum((a0 + a1) + (a2 + a3))
        return jnp.where(lane == c, total, sums)

    sums0 = lax.fori_loop(0, N // CHUNK, chunk_body, zero)
    scan0 = plsc.cumsum(sums0)
    mask0 = scan0 < t_r
    c_star = jnp.sum(mask0.astype(jnp.int32))
    s_before = jnp.sum(jnp.where(mask0, sums0, 0.0))
    count = c_star * CHUNK
    base1 = jnp.minimum(c_star, N // CHUNK - 1) * CHUNK

    def sub_body(s, sums):
        base = base1 + s * SUB

        def vbody(i, accs):
            a0, a1 = accs
            off = base + i * (2 * L)
            a0 = a0 + rowbuf[pl.ds(off + 0 * L, L)]
            a1 = a1 + rowbuf[pl.ds(off + 1 * L, L)]
            return (a0, a1)

        a0, a1 = lax.fori_loop(0, SUB // (2 * L), vbody, (zero, zero))
        total = jnp.sum(a0 + a1)
        return jnp.where(lane == s, total, sums)

    sums1 = lax.fori_loop(0, CHUNK // SUB, sub_body, zero)
    scan1 = plsc.cumsum(sums1)
    mask1 = (s_before + scan1) < t_r
    s_star = jnp.sum(mask1.astype(jnp.int32))
    s_before = s_before + jnp.sum(jnp.where(mask1, sums1, 0.0))
    count = count + s_star * SUB
    base2 = base1 + jnp.minimum(s_star, CHUNK // SUB - 1) * SUB

    def fine_body(i, carry):
        cnt, s_run = carry
        v = rowbuf[pl.ds(base2 + i * L, L)]
        sc = plsc.cumsum(v)
        cnt = cnt + jnp.sum(((s_run + sc) < t_r).astype(jnp.int32))
        s_run = s_run + jnp.sum(v)
        return (cnt, s_run)

    count, _ = lax.fori_loop(0, SUB // L, fine_body, (count, s_before))
    return count


def _sc_body(theta_hbm, t_hbm, y_hbm, out_hbm,
             t_v, rb0, rb1, val_v, sem0, sem1, sem_y):
    wid = lax.axis_index("s") * NC + lax.axis_index("c")
    row_base = wid * ROWS_PER_W
    lane = lax.iota(jnp.int32, L)

    pltpu.sync_copy(t_hbm, t_v.at[pl.ds(0, B)])

    bufs = (rb0, rb1)
    sems = (sem0, sem1)
    cps = [None, None]
    cps[0] = pltpu.make_async_copy(theta_hbm.at[row_base], rb0, sem0)
    cps[0].start()

    idx_vec = jnp.zeros((L,), jnp.int32)
    for r in range(ROWS_PER_W):
        cur = r % 2
        cps[cur].wait()
        if r + 1 < ROWS_PER_W:
            nxt = (r + 1) % 2
            cps[nxt] = pltpu.make_async_copy(
                theta_hbm.at[row_base + r + 1], bufs[nxt], sems[nxt])
            cps[nxt].start()
        t_r = t_v[pl.ds(row_base + r, L)][0]
        count = _count_below(bufs[cur], t_r)
        idx = jnp.clip(count, 0, N - 1)
        idx_vec = jnp.where(lane == r, idx, idx_vec)

    pltpu.async_copy(y_hbm.at[idx_vec], val_v, sem_y).wait()
    pltpu.sync_copy(val_v, out_hbm.at[wid])


@jax.jit
def _sc_quantile(theta, t, Y_train):
    mesh = plsc.VectorSubcoreMesh(core_axis_name="c", subcore_axis_name="s")
    out2d = pl.kernel(
        _sc_body,
        out_type=jax.ShapeDtypeStruct((NW, L), jnp.float32),
        mesh=mesh,
        compiler_params=pltpu.CompilerParams(needs_layout_passes=False),
        scratch_types=[
            pltpu.VMEM((B + L,), jnp.float32),
            pltpu.VMEM((N,), jnp.float32),
            pltpu.VMEM((N,), jnp.float32),
            pltpu.VMEM((L,), jnp.float32),
            pltpu.SemaphoreType.DMA,
            pltpu.SemaphoreType.DMA,
            pltpu.SemaphoreType.DMA,
        ],
    )(theta, t, Y_train)
    return out2d[:, :ROWS_PER_W].reshape(B)


def kernel(theta, t, Y_train):
    return _sc_quantile(theta, t, Y_train)
